# Initial kernel scaffold; baseline (speedup 1.0000x reference)
import jax
import jax.numpy as jnp
from jax.experimental import pallas as pl


def kernel(x, e, pos, neg, W0, b0, W1, b1, W2, b2, W3, b3, W4, b4, Wc, bc):
    raise NotImplementedError("write your pallas kernel here")


# speedup vs baseline: 9.0141x; 9.0141x over previous
import functools

import jax
import jax.numpy as jnp
from jax import lax
from jax.experimental import pallas as pl
from jax.experimental.pallas import tpu as pltpu
from jax.experimental.pallas import tpu_sc as plsc

N_NODES = 10000
NP = 10240
EDGES = 330000
ECHUNK = 4096
EALLOC = EDGES + ECHUNK + 16
NT = 32
ROWS_PER_TILE = NP // NT
PANEL = ROWS_PER_TILE * NP
ZCH = 32768
NZC = PANEL // ZCH
DCH = 6000
NDC = EDGES // DCH
JUNK_COL = NP - 16


def _rsqrt16(x):
    i = lax.bitcast_convert_type(x, jnp.int32)
    y = lax.bitcast_convert_type(
        jnp.int32(0x5F3759DF) - lax.shift_right_logical(i, 1), jnp.float32)
    for _ in range(3):
        y = y * (1.5 - 0.5 * x * y * y)
    return y


def _scalar_at(vref, i):
    acc = jnp.int32(0)
    lanes = jnp.arange(16, dtype=jnp.int32)
    for v in range(3):
        vec = vref[pl.ds(v * 16, 16)]
        acc += jnp.sum(jnp.where(lanes + v * 16 == i, vec, 0))
    return acc


def _adj_body(src_hbm, dst_hbm, cf_hbm, off_hbm, a_hbm,
              zbuf, dstb, deg_v, dinv_v, offv, sbuf, dbuf, cfb,
              abuf, vbuf, zsem, ssem):
    wid = lax.axis_index("s") * 2 + lax.axis_index("c")
    zeros16 = jnp.zeros((16,), jnp.float32)
    ones16 = jnp.ones((16,), jnp.float32)
    lanes = jnp.arange(16, dtype=jnp.int32)

    def zb(i, _):
        zbuf[pl.ds(i * 16, 16)] = zeros16
        return 0
    lax.fori_loop(0, ZCH // 16, zb, 0)
    base = wid * PANEL
    copies = [
        pltpu.async_copy(zbuf, a_hbm.at[pl.ds(base + k * ZCH, ZCH)], zsem)
        for k in range(NZC)
    ]

    def db(i, _):
        deg_v[pl.ds(i * 16, 16)] = zeros16
        return 0
    lax.fori_loop(0, N_NODES // 16, db, 0)

    def dchunk(k, _):
        doff = pl.multiple_of(k * DCH, 8)
        pltpu.sync_copy(dst_hbm.at[pl.ds(doff, DCH)], dstb)
        def dvec(v, _):
            idx = dstb[pl.ds(v * 16, 16)]
            plsc.addupdate_scatter(deg_v, [idx], ones16)
            return 0
        lax.fori_loop(0, DCH // 16, dvec, 0)
        return 0
    lax.fori_loop(0, NDC, dchunk, 0)

    def vb(i, _):
        dinv_v[pl.ds(i * 16, 16)] = _rsqrt16(deg_v[pl.ds(i * 16, 16)])
        return 0
    lax.fori_loop(0, N_NODES // 16, vb, 0)

    pltpu.sync_copy(off_hbm, offv)
    off_lo = _scalar_at(offv, wid)
    off_hi = _scalar_at(offv, wid + 1)
    start0 = off_lo & jnp.int32(~15)
    nch = lax.shift_right_logical(off_hi - start0 + (ECHUNK - 1),
                                  jnp.int32(12))

    for c in copies:
        c.wait()

    junk = wid * ROWS_PER_TILE * NP + JUNK_COL + lanes

    def schunk(k, _):
        cbase = start0 + k * ECHUNK
        coff = pl.multiple_of(cbase, 16)
        pltpu.sync_copy(src_hbm.at[pl.ds(coff, ECHUNK)], sbuf)
        pltpu.sync_copy(dst_hbm.at[pl.ds(coff, ECHUNK)], dbuf)
        pltpu.sync_copy(cf_hbm.at[pl.ds(coff, ECHUNK)], cfb)

        def sgroup(g, _):
            for j in range(8):
                o = g * 128 + j * 16
                sv = sbuf[pl.ds(o, 16)]
                dv = dbuf[pl.ds(o, 16)]
                cfv = cfb[pl.ds(o, 16)]
                ds_ = plsc.load_gather(dinv_v, [sv])
                dd_ = plsc.load_gather(dinv_v, [dv])
                val = cfv * ds_ * dd_
                eidx = cbase + o + lanes
                ok = (eidx >= off_lo) & (eidx < off_hi) & (cfv > 0.0)
                addr = jnp.where(ok, dv * NP + sv, junk)
                abuf[pl.ds(j * 16, 16)] = addr
                vbuf[pl.ds(j * 16, 16)] = val
            pltpu.async_copy(vbuf, a_hbm.at[abuf], ssem).wait()
            return 0
        lax.fori_loop(0, ECHUNK // 128, sgroup, 0)
        return 0
    lax.fori_loop(0, nch, schunk, 0)


def _build_adj_sc(srcS, dstS, cfS, offs):
    mesh = plsc.VectorSubcoreMesh(core_axis_name="c", subcore_axis_name="s")
    f = pl.kernel(
        _adj_body,
        out_type=jax.ShapeDtypeStruct((NP * NP,), jnp.float32),
        mesh=mesh,
        scratch_types=[
            pltpu.VMEM((ZCH,), jnp.float32),
            pltpu.VMEM((DCH,), jnp.int32),
            pltpu.VMEM((N_NODES,), jnp.float32),
            pltpu.VMEM((N_NODES,), jnp.float32),
            pltpu.VMEM((48,), jnp.int32),
            pltpu.VMEM((ECHUNK,), jnp.int32),
            pltpu.VMEM((ECHUNK,), jnp.int32),
            pltpu.VMEM((ECHUNK,), jnp.float32),
            pltpu.VMEM((128,), jnp.int32),
            pltpu.VMEM((128,), jnp.float32),
            pltpu.SemaphoreType.DMA,
            pltpu.SemaphoreType.DMA,
        ],
        compiler_params=pltpu.CompilerParams(needs_layout_passes=False),
    )
    return f(srcS, dstS, cfS, offs)


GB = 262144
GROWS = GB // NT
GCH = 128


def _gather_body(z_hbm, idx_hbm, out_hbm, ibuf, rows0, rows1, sem):
    wid = lax.axis_index("s") * 2 + lax.axis_index("c")
    base = pl.multiple_of(wid * GROWS, 128)
    pltpu.sync_copy(idx_hbm.at[pl.ds(base, GROWS)], ibuf)
    bufs = [rows0, rows1]
    nch = GROWS // GCH
    for k in range(nch):
        rb = bufs[k % 2]
        pltpu.async_copy(z_hbm.at[ibuf.at[pl.ds(k * GCH, GCH)]], rb, sem).wait()
        pltpu.sync_copy(rb, out_hbm.at[pl.ds(base + k * GCH, GCH), :])


def _gather_sc(z, idx):
    mesh = plsc.VectorSubcoreMesh(core_axis_name="c", subcore_axis_name="s")
    f = pl.kernel(
        _gather_body,
        out_type=jax.ShapeDtypeStruct((GB, 128), jnp.float32),
        mesh=mesh,
        scratch_types=[
            pltpu.VMEM((GROWS,), jnp.int32),
            pltpu.VMEM((GCH, 128), jnp.float32),
            pltpu.VMEM((GCH, 128), jnp.float32),
            pltpu.SemaphoreType.DMA,
        ],
        compiler_params=pltpu.CompilerParams(needs_layout_passes=False),
    )
    return f(z, idx)


def _mm_kernel(x_ref, w_ref, b_ref, o_ref, acc_ref, *, nk, relu, mask_rows):
    k = pl.program_id(1)

    @pl.when(k == 0)
    def _():
        acc_ref[...] = jnp.zeros_like(acc_ref)

    acc_ref[...] += jnp.dot(x_ref[...], w_ref[...],
                            preferred_element_type=jnp.float32)

    @pl.when(k == nk - 1)
    def _():
        res = acc_ref[...] + b_ref[...]
        if relu:
            res = jnp.maximum(res, 0.0)
        if mask_rows:
            i = pl.program_id(0)
            mb, n = acc_ref.shape
            row = i * mb + lax.broadcasted_iota(jnp.int32, (mb, n), 0)
            res = jnp.where(row < N_NODES, res, 0.0)
        o_ref[...] = res


def _mm(x, w, b=None, relu=False, mask_rows=False, mb=2048, kb=1024):
    m, kdim = x.shape
    n = w.shape[1]
    if b is None:
        b = jnp.zeros((1, n), dtype=jnp.float32)
    else:
        b = b.reshape(1, n)
    kb = min(kb, kdim)
    nk = kdim // kb
    grid = (m // mb, nk)
    return pl.pallas_call(
        functools.partial(_mm_kernel, nk=nk, relu=relu, mask_rows=mask_rows),
        grid=grid,
        in_specs=[
            pl.BlockSpec((mb, kb), lambda i, j: (i, j)),
            pl.BlockSpec((kb, n), lambda i, j: (j, 0)),
            pl.BlockSpec((1, n), lambda i, j: (0, 0)),
        ],
        out_specs=pl.BlockSpec((mb, n), lambda i, j: (i, 0)),
        out_shape=jax.ShapeDtypeStruct((m, n), jnp.float32),
        scratch_shapes=[pltpu.VMEM((mb, n), jnp.float32)],
    )(x, w, b)


def _pair_kernel(a_ref, b_ref, o_ref):
    s = jnp.sum(a_ref[...] * b_ref[...], axis=1)
    o_ref[...] = 1.0 / (1.0 + jnp.exp(-s))


def _pair_preds(za, zb):
    m = za.shape[0]
    blk = 8192
    return pl.pallas_call(
        _pair_kernel,
        grid=(m // blk,),
        in_specs=[
            pl.BlockSpec((blk, 128), lambda i: (i, 0)),
            pl.BlockSpec((blk, 128), lambda i: (i, 0)),
        ],
        out_specs=pl.BlockSpec((blk,), lambda i: (i,)),
        out_shape=jax.ShapeDtypeStruct((m,), jnp.float32),
    )(za, zb)


def kernel(x, e, pos, neg, W0, b0, W1, b1, W2, b2, W3, b3, W4, b4, Wc, bc):
    loop = jnp.arange(N_NODES, dtype=e.dtype)
    srcA = jnp.concatenate([e[0], loop])
    dstA = jnp.concatenate([e[1], loop])
    key = dstA.astype(jnp.int32) * NP + srcA.astype(jnp.int32)
    ks = jnp.sort(key)
    dstS = ks // NP
    srcS = ks - dstS * NP
    posi = jnp.arange(EDGES, dtype=jnp.int32)
    isf = jnp.concatenate([jnp.array([True]), ks[1:] != ks[:-1]])
    firsts = jnp.where(isf, posi, jnp.int32(EDGES))
    nxt = lax.cummin(firsts[::-1])[::-1]
    nxt_after = jnp.concatenate([nxt[1:], jnp.array([EDGES], jnp.int32)])
    cf = jnp.where(isf, (nxt_after - posi).astype(jnp.float32), 0.0)
    rowb = jnp.arange(0, NP + ROWS_PER_TILE, ROWS_PER_TILE, dtype=jnp.int32)
    offs = jnp.searchsorted(dstS, rowb[:33], side="left").astype(jnp.int32)
    offs = jnp.concatenate([offs, jnp.zeros((15,), jnp.int32)])
    pad = EALLOC - EDGES
    srcP = jnp.concatenate([srcS, jnp.zeros((pad,), jnp.int32)])
    dstP = jnp.concatenate([dstS, jnp.zeros((pad,), jnp.int32)])
    cfP = jnp.concatenate([cf, jnp.zeros((pad,), jnp.float32)])

    A = _build_adj_sc(srcP, dstP, cfP, offs).reshape(NP, NP)

    z = jnp.zeros((NP, 128), jnp.float32).at[:N_NODES].set(x)
    for W, b in ((W0, b0), (W1, b1), (W2, b2), (W3, b3), (W4, b4)):
        h = _mm(z, W)
        z = _mm(A, h, b=b, relu=True, mask_rows=True)

    Wcp = jnp.zeros((128, 128), jnp.float32).at[:, :4].set(Wc)
    bcp = jnp.zeros((128,), jnp.float32).at[:4].set(bc)
    logits = _mm(z, Wcp, b=bcp)[:N_NODES, :4]

    gidx = jnp.concatenate([pos[0], neg[0], pos[1], neg[1]]).astype(jnp.int32)
    rows = _gather_sc(z, gidx)
    preds = _pair_preds(rows[: GB // 2], rows[GB // 2:])
    return (z[:N_NODES], logits, preds)


# speedup vs baseline: 9.2420x; 1.0253x over previous
import functools

import jax
import jax.numpy as jnp
from jax import lax
from jax.experimental import pallas as pl
from jax.experimental.pallas import tpu as pltpu
from jax.experimental.pallas import tpu_sc as plsc

N_NODES = 10000
NP = 10240
EDGES = 330000
ECHUNK = 4096
EALLOC = EDGES + ECHUNK + 16
NT = 32
ROWS_PER_TILE = NP // NT
PANEL = ROWS_PER_TILE * NP
ZCH = 32768
NZC = PANEL // ZCH
DCH = 6000
NDC = EDGES // DCH
JUNK_COL = NP - 16


def _rsqrt16(x):
    i = lax.bitcast_convert_type(x, jnp.int32)
    y = lax.bitcast_convert_type(
        jnp.int32(0x5F3759DF) - lax.shift_right_logical(i, 1), jnp.float32)
    for _ in range(3):
        y = y * (1.5 - 0.5 * x * y * y)
    return y


def _scalar_at(vref, i):
    acc = jnp.int32(0)
    lanes = jnp.arange(16, dtype=jnp.int32)
    for v in range(3):
        vec = vref[pl.ds(v * 16, 16)]
        acc += jnp.sum(jnp.where(lanes + v * 16 == i, vec, 0))
    return acc


def _adj_body(src_hbm, dst_hbm, cf_hbm, off_hbm, a_hbm,
              zbuf, dstb, deg_v, dinv_v, offv, sbuf, dbuf, cfb,
              abuf, vbuf, zsem, ssem):
    wid = lax.axis_index("s") * 2 + lax.axis_index("c")
    zeros16 = jnp.zeros((16,), jnp.float32)
    ones16 = jnp.ones((16,), jnp.float32)
    lanes = jnp.arange(16, dtype=jnp.int32)

    def zb(i, _):
        zbuf[pl.ds(i * 16, 16)] = zeros16
        return 0
    lax.fori_loop(0, ZCH // 16, zb, 0)
    base = wid * PANEL
    copies = [
        pltpu.async_copy(zbuf, a_hbm.at[pl.ds(base + k * ZCH, ZCH)], zsem)
        for k in range(NZC)
    ]

    def db(i, _):
        deg_v[pl.ds(i * 16, 16)] = zeros16
        return 0
    lax.fori_loop(0, N_NODES // 16, db, 0)

    def dchunk(k, _):
        doff = pl.multiple_of(k * DCH, 8)
        pltpu.sync_copy(dst_hbm.at[pl.ds(doff, DCH)], dstb)
        def dvec(v, _):
            idx = dstb[pl.ds(v * 16, 16)]
            plsc.addupdate_scatter(deg_v, [idx], ones16)
            return 0
        lax.fori_loop(0, DCH // 16, dvec, 0)
        return 0
    lax.fori_loop(0, NDC, dchunk, 0)

    def vb(i, _):
        dinv_v[pl.ds(i * 16, 16)] = _rsqrt16(deg_v[pl.ds(i * 16, 16)])
        return 0
    lax.fori_loop(0, N_NODES // 16, vb, 0)

    pltpu.sync_copy(off_hbm, offv)
    off_lo = _scalar_at(offv, wid)
    off_hi = _scalar_at(offv, wid + 1)
    start0 = off_lo & jnp.int32(~15)
    nch = lax.shift_right_logical(off_hi - start0 + (ECHUNK - 1),
                                  jnp.int32(12))

    for c in copies:
        c.wait()

    junk = wid * ROWS_PER_TILE * NP + JUNK_COL + lanes

    def schunk(k, _):
        cbase = start0 + k * ECHUNK
        coff = pl.multiple_of(cbase, 16)
        pltpu.sync_copy(src_hbm.at[pl.ds(coff, ECHUNK)], sbuf)
        pltpu.sync_copy(dst_hbm.at[pl.ds(coff, ECHUNK)], dbuf)
        pltpu.sync_copy(cf_hbm.at[pl.ds(coff, ECHUNK)], cfb)

        def sgroup(g, _):
            for j in range(8):
                o = g * 128 + j * 16
                sv = sbuf[pl.ds(o, 16)]
                dv = dbuf[pl.ds(o, 16)]
                cfv = cfb[pl.ds(o, 16)]
                ds_ = plsc.load_gather(dinv_v, [sv])
                dd_ = plsc.load_gather(dinv_v, [dv])
                val = cfv * ds_ * dd_
                eidx = cbase + o + lanes
                ok = (eidx >= off_lo) & (eidx < off_hi) & (cfv > 0.0)
                addr = jnp.where(ok, dv * NP + sv, junk)
                abuf[pl.ds(j * 16, 16)] = addr
                vbuf[pl.ds(j * 16, 16)] = val
            pltpu.async_copy(vbuf, a_hbm.at[abuf], ssem).wait()
            return 0
        lax.fori_loop(0, ECHUNK // 128, sgroup, 0)
        return 0
    lax.fori_loop(0, nch, schunk, 0)


def _build_adj_sc(srcS, dstS, cfS, offs):
    mesh = plsc.VectorSubcoreMesh(core_axis_name="c", subcore_axis_name="s")
    f = pl.kernel(
        _adj_body,
        out_type=jax.ShapeDtypeStruct((NP * NP,), jnp.float32),
        mesh=mesh,
        scratch_types=[
            pltpu.VMEM((ZCH,), jnp.float32),
            pltpu.VMEM((DCH,), jnp.int32),
            pltpu.VMEM((N_NODES,), jnp.float32),
            pltpu.VMEM((N_NODES,), jnp.float32),
            pltpu.VMEM((48,), jnp.int32),
            pltpu.VMEM((ECHUNK,), jnp.int32),
            pltpu.VMEM((ECHUNK,), jnp.int32),
            pltpu.VMEM((ECHUNK,), jnp.float32),
            pltpu.VMEM((128,), jnp.int32),
            pltpu.VMEM((128,), jnp.float32),
            pltpu.SemaphoreType.DMA,
            pltpu.SemaphoreType.DMA,
        ],
        compiler_params=pltpu.CompilerParams(needs_layout_passes=False),
    )
    return f(srcS, dstS, cfS, offs)


GB = 262144
GROWS = GB // NT
GCH = 128


def _gather_body(z_hbm, idx_hbm, out_hbm, ibuf, rows0, rows1, sem):
    wid = lax.axis_index("s") * 2 + lax.axis_index("c")
    base = pl.multiple_of(wid * GROWS, 128)
    pltpu.sync_copy(idx_hbm.at[pl.ds(base, GROWS)], ibuf)
    bufs = [rows0, rows1]
    nch = GROWS // GCH
    for k in range(nch):
        rb = bufs[k % 2]
        pltpu.async_copy(z_hbm.at[ibuf.at[pl.ds(k * GCH, GCH)]], rb, sem).wait()
        pltpu.sync_copy(rb, out_hbm.at[pl.ds(base + k * GCH, GCH), :])


def _gather_sc(z, idx):
    mesh = plsc.VectorSubcoreMesh(core_axis_name="c", subcore_axis_name="s")
    f = pl.kernel(
        _gather_body,
        out_type=jax.ShapeDtypeStruct((GB, 128), jnp.float32),
        mesh=mesh,
        scratch_types=[
            pltpu.VMEM((GROWS,), jnp.int32),
            pltpu.VMEM((GCH, 128), jnp.float32),
            pltpu.VMEM((GCH, 128), jnp.float32),
            pltpu.SemaphoreType.DMA,
        ],
        compiler_params=pltpu.CompilerParams(needs_layout_passes=False),
    )
    return f(z, idx)


def _mma_kernel(x_ref, w_ref, b_ref, o_ref, a16_ref, acc_ref, *, nk, first):
    k = pl.program_id(1)

    @pl.when(k == 0)
    def _():
        acc_ref[...] = jnp.zeros_like(acc_ref)

    if first:
        xb = x_ref[...].astype(jnp.bfloat16)
        a16_ref[...] = xb
    else:
        xb = x_ref[...]
    acc_ref[...] += jnp.dot(xb, w_ref[...].astype(jnp.bfloat16),
                            preferred_element_type=jnp.float32)

    @pl.when(k == nk - 1)
    def _():
        res = jnp.maximum(acc_ref[...] + b_ref[...], 0.0)
        i = pl.program_id(0)
        mb, n = acc_ref.shape
        row = i * mb + lax.broadcasted_iota(jnp.int32, (mb, n), 0)
        o_ref[...] = jnp.where(row < N_NODES, res, 0.0)


def _mm_a(x, h, b, first, mb=2048, kb=1024):
    n = h.shape[1]
    b = b.reshape(1, n)
    nk = NP // kb
    grid = (NP // mb, nk)
    out_shape = [jax.ShapeDtypeStruct((NP, n), jnp.float32)]
    out_specs = [pl.BlockSpec((mb, n), lambda i, j: (i, 0))]
    if first:
        out_shape.append(jax.ShapeDtypeStruct((NP, NP), jnp.bfloat16))
        out_specs.append(pl.BlockSpec((mb, kb), lambda i, j: (i, j)))
    else:
        out_shape.append(jax.ShapeDtypeStruct((8, 128), jnp.bfloat16))
        out_specs.append(pl.BlockSpec((8, 128), lambda i, j: (0, 0)))
    res = pl.pallas_call(
        functools.partial(_mma_kernel, nk=nk, first=first),
        grid=grid,
        in_specs=[
            pl.BlockSpec((mb, kb), lambda i, j: (i, j)),
            pl.BlockSpec((kb, n), lambda i, j: (j, 0)),
            pl.BlockSpec((1, n), lambda i, j: (0, 0)),
        ],
        out_specs=out_specs,
        out_shape=out_shape,
        scratch_shapes=[pltpu.VMEM((mb, n), jnp.float32)],
    )(x, h, b)
    return res


def _mm_kernel(x_ref, w_ref, b_ref, o_ref, acc_ref, *, nk, relu, mask_rows):
    k = pl.program_id(1)

    @pl.when(k == 0)
    def _():
        acc_ref[...] = jnp.zeros_like(acc_ref)

    acc_ref[...] += jnp.dot(x_ref[...], w_ref[...],
                            preferred_element_type=jnp.float32)

    @pl.when(k == nk - 1)
    def _():
        res = acc_ref[...] + b_ref[...]
        if relu:
            res = jnp.maximum(res, 0.0)
        if mask_rows:
            i = pl.program_id(0)
            mb, n = acc_ref.shape
            row = i * mb + lax.broadcasted_iota(jnp.int32, (mb, n), 0)
            res = jnp.where(row < N_NODES, res, 0.0)
        o_ref[...] = res


def _mm(x, w, b=None, relu=False, mask_rows=False, mb=2048, kb=1024):
    m, kdim = x.shape
    n = w.shape[1]
    if b is None:
        b = jnp.zeros((1, n), dtype=jnp.float32)
    else:
        b = b.reshape(1, n)
    kb = min(kb, kdim)
    nk = kdim // kb
    grid = (m // mb, nk)
    return pl.pallas_call(
        functools.partial(_mm_kernel, nk=nk, relu=relu, mask_rows=mask_rows),
        grid=grid,
        in_specs=[
            pl.BlockSpec((mb, kb), lambda i, j: (i, j)),
            pl.BlockSpec((kb, n), lambda i, j: (j, 0)),
            pl.BlockSpec((1, n), lambda i, j: (0, 0)),
        ],
        out_specs=pl.BlockSpec((mb, n), lambda i, j: (i, 0)),
        out_shape=jax.ShapeDtypeStruct((m, n), jnp.float32),
        scratch_shapes=[pltpu.VMEM((mb, n), jnp.float32)],
    )(x, w, b)


def _pair_kernel(a_ref, b_ref, o_ref):
    s = jnp.sum(a_ref[...] * b_ref[...], axis=1)
    o_ref[...] = 1.0 / (1.0 + jnp.exp(-s))


def _pair_preds(za, zb):
    m = za.shape[0]
    blk = 8192
    return pl.pallas_call(
        _pair_kernel,
        grid=(m // blk,),
        in_specs=[
            pl.BlockSpec((blk, 128), lambda i: (i, 0)),
            pl.BlockSpec((blk, 128), lambda i: (i, 0)),
        ],
        out_specs=pl.BlockSpec((blk,), lambda i: (i,)),
        out_shape=jax.ShapeDtypeStruct((m,), jnp.float32),
    )(za, zb)


def kernel(x, e, pos, neg, W0, b0, W1, b1, W2, b2, W3, b3, W4, b4, Wc, bc):
    loop = jnp.arange(N_NODES, dtype=e.dtype)
    srcA = jnp.concatenate([e[0], loop])
    dstA = jnp.concatenate([e[1], loop])
    key = dstA.astype(jnp.int32) * NP + srcA.astype(jnp.int32)
    ks = jnp.sort(key)
    dstS = ks // NP
    srcS = ks - dstS * NP
    posi = jnp.arange(EDGES, dtype=jnp.int32)
    isf = jnp.concatenate([jnp.array([True]), ks[1:] != ks[:-1]])
    firsts = jnp.where(isf, posi, jnp.int32(EDGES))
    nxt = lax.cummin(firsts[::-1])[::-1]
    nxt_after = jnp.concatenate([nxt[1:], jnp.array([EDGES], jnp.int32)])
    cf = jnp.where(isf, (nxt_after - posi).astype(jnp.float32), 0.0)
    rowb = jnp.arange(0, NP + ROWS_PER_TILE, ROWS_PER_TILE, dtype=jnp.int32)
    offs = jnp.searchsorted(dstS, rowb[:33], side="left").astype(jnp.int32)
    offs = jnp.concatenate([offs, jnp.zeros((15,), jnp.int32)])
    pad = EALLOC - EDGES
    srcP = jnp.concatenate([srcS, jnp.zeros((pad,), jnp.int32)])
    dstP = jnp.concatenate([dstS, jnp.zeros((pad,), jnp.int32)])
    cfP = jnp.concatenate([cf, jnp.zeros((pad,), jnp.float32)])

    A = _build_adj_sc(srcP, dstP, cfP, offs).reshape(NP, NP)

    z = jnp.zeros((NP, 128), jnp.float32).at[:N_NODES].set(x)
    A16 = None
    for li, (W, b) in enumerate(((W0, b0), (W1, b1), (W2, b2),
                                 (W3, b3), (W4, b4))):
        h = _mm(z, W)
        if li == 0:
            z, A16 = _mm_a(A, h, b, first=True)
        else:
            z, _ = _mm_a(A16, h, b, first=False)

    Wcp = jnp.zeros((128, 128), jnp.float32).at[:, :4].set(Wc)
    bcp = jnp.zeros((128,), jnp.float32).at[:4].set(bc)
    logits = _mm(z, Wcp, b=bcp)[:N_NODES, :4]

    gidx = jnp.concatenate([pos[0], neg[0], pos[1], neg[1]]).astype(jnp.int32)
    rows = _gather_sc(z, gidx)
    preds = _pair_preds(rows[: GB // 2], rows[GB // 2:])
    return (z[:N_NODES], logits, preds)


# speedup vs baseline: 10.3475x; 1.1196x over previous
import functools

import jax
import jax.numpy as jnp
from jax import lax
from jax.experimental import pallas as pl
from jax.experimental.pallas import tpu as pltpu
from jax.experimental.pallas import tpu_sc as plsc

N_NODES = 10000
NP = 10240
EDGES = 330000
ECHUNK = 4096
EALLOC = EDGES + ECHUNK + 16
NT = 32
ROWS_PER_TILE = NP // NT
PANEL = ROWS_PER_TILE * NP
ZCH = 32768
NZC = PANEL // ZCH
ESUB = 20640
DROWS = 80
JUNK_COL = NP - 16


def _rsqrt16(x):
    i = lax.bitcast_convert_type(x, jnp.int32)
    y = lax.bitcast_convert_type(
        jnp.int32(0x5F3759DF) - lax.shift_right_logical(i, 1), jnp.float32)
    for _ in range(3):
        y = y * (1.5 - 0.5 * x * y * y)
    return y


def _scalar_at(vref, i):
    acc = jnp.int32(0)
    lanes = jnp.arange(16, dtype=jnp.int32)
    for v in range(3):
        vec = vref[pl.ds(v * 16, 16)]
        acc += jnp.sum(jnp.where(lanes + v * 16 == i, vec, 0))
    return acc


def _adj_body(src_hbm, dst_hbm, cf_hbm, off_hbm, a_hbm,
              zbuf, dstb, deg2, dinv_v, offv, sbuf, dbuf, cfb,
              abufs, vbufs, ridx, shared, zsem, ssem):
    cid = lax.axis_index("c")
    sid = lax.axis_index("s")
    wid = sid * 2 + cid
    zeros16 = jnp.zeros((16,), jnp.float32)
    ones16 = jnp.ones((16,), jnp.float32)
    lanes = jnp.arange(16, dtype=jnp.int32)

    def zb(i, _):
        zbuf[pl.ds(i * 16, 16)] = zeros16
        return 0
    lax.fori_loop(0, ZCH // 16, zb, 0)
    base = wid * PANEL
    copies = [
        pltpu.async_copy(zbuf, a_hbm.at[pl.ds(base + k * ZCH, ZCH)], zsem)
        for k in range(NZC)
    ]

    def db(v, _):
        r = jnp.full((16,), lax.shift_right_logical(v, 3), jnp.int32)
        c = (v & 7) * 16 + lanes
        plsc.store_scatter(deg2, [r, c], zeros16)
        return 0
    lax.fori_loop(0, DROWS * 8, db, 0)

    @pl.when(sid == 0)
    def _():
        pltpu.sync_copy(deg2, shared)
    plsc.subcore_barrier()

    dbase = pl.multiple_of(sid * ESUB, 8)
    pltpu.sync_copy(dst_hbm.at[pl.ds(dbase, ESUB)], dstb)
    nvec = jnp.where(sid < 15, ESUB // 16, (EDGES - 15 * ESUB) // 16)

    def dvec(v, _):
        idx = dstb[pl.ds(v * 16, 16)]
        r = lax.shift_right_logical(idx, 7)
        c = idx & 127
        plsc.addupdate_scatter(deg2, [r, c], ones16)
        return 0
    lax.fori_loop(0, nvec, dvec, 0)

    for j in range(DROWS // 16):
        ridx[pl.ds(j * 16, 16)] = j * 16 + lanes
    pltpu.sync_copy(deg2, shared.at[ridx], add=True)
    plsc.subcore_barrier()
    pltpu.sync_copy(shared, deg2)

    def vb(v, _):
        rv = jnp.full((16,), lax.shift_right_logical(v, 3), jnp.int32)
        cv = (v & 7) * 16 + lanes
        vals = plsc.load_gather(deg2, [rv, cv])
        dinv_v[pl.ds(v * 16, 16)] = _rsqrt16(vals)
        return 0
    lax.fori_loop(0, DROWS * 8, vb, 0)

    pltpu.sync_copy(off_hbm, offv)
    off_lo = _scalar_at(offv, wid)
    off_hi = _scalar_at(offv, wid + 1)
    start0 = off_lo & jnp.int32(~15)
    nch = lax.shift_right_logical(off_hi - start0 + (ECHUNK - 1),
                                  jnp.int32(12))

    for c in copies:
        c.wait()

    junk = wid * ROWS_PER_TILE * NP + JUNK_COL + lanes

    def schunk(k, _):
        cbase = start0 + k * ECHUNK
        coff = pl.multiple_of(cbase, 16)
        pltpu.sync_copy(src_hbm.at[pl.ds(coff, ECHUNK)], sbuf)
        pltpu.sync_copy(dst_hbm.at[pl.ds(coff, ECHUNK)], dbuf)
        pltpu.sync_copy(cf_hbm.at[pl.ds(coff, ECHUNK)], cfb)

        for sc in range(ECHUNK // 1024):
            handles = []
            for g in range(8):
                ab, vb_ = abufs[g], vbufs[g]
                for j in range(8):
                    o = sc * 1024 + g * 128 + j * 16
                    sv = sbuf[pl.ds(o, 16)]
                    dv = dbuf[pl.ds(o, 16)]
                    cfv = cfb[pl.ds(o, 16)]
                    ds_ = plsc.load_gather(dinv_v, [sv])
                    dd_ = plsc.load_gather(dinv_v, [dv])
                    val = cfv * ds_ * dd_
                    eidx = cbase + o + lanes
                    ok = (eidx >= off_lo) & (eidx < off_hi) & (cfv > 0.0)
                    addr = jnp.where(ok, dv * NP + sv, junk)
                    ab[pl.ds(j * 16, 16)] = addr
                    vb_[pl.ds(j * 16, 16)] = val
                handles.append(
                    pltpu.async_copy(vb_, a_hbm.at[ab], ssem))
            for h in handles:
                h.wait()
        return 0
    lax.fori_loop(0, nch, schunk, 0)


def _build_adj_sc(srcS, dstS, cfS, offs):
    mesh = plsc.VectorSubcoreMesh(core_axis_name="c", subcore_axis_name="s")
    f = pl.kernel(
        _adj_body,
        out_type=jax.ShapeDtypeStruct((NP * NP,), jnp.float32),
        mesh=mesh,
        scratch_types=[
            pltpu.VMEM((ZCH,), jnp.float32),
            pltpu.VMEM((ESUB,), jnp.int32),
            pltpu.VMEM((DROWS, 128), jnp.float32),
            pltpu.VMEM((NP,), jnp.float32),
            pltpu.VMEM((48,), jnp.int32),
            pltpu.VMEM((ECHUNK,), jnp.int32),
            pltpu.VMEM((ECHUNK,), jnp.int32),
            pltpu.VMEM((ECHUNK,), jnp.float32),
            [pltpu.VMEM((128,), jnp.int32) for _ in range(8)],
            [pltpu.VMEM((128,), jnp.float32) for _ in range(8)],
            pltpu.VMEM((DROWS,), jnp.int32),
            pltpu.VMEM_SHARED((DROWS, 128), jnp.float32),
            pltpu.SemaphoreType.DMA,
            pltpu.SemaphoreType.DMA,
        ],
        compiler_params=pltpu.CompilerParams(needs_layout_passes=False),
    )
    return f(srcS, dstS, cfS, offs)


GB = 262144
GROWS = GB // NT
GCH = 128


def _gather_body(z_hbm, idx_hbm, out_hbm, ibuf, rows0, rows1, sem):
    wid = lax.axis_index("s") * 2 + lax.axis_index("c")
    base = pl.multiple_of(wid * GROWS, 128)
    pltpu.sync_copy(idx_hbm.at[pl.ds(base, GROWS)], ibuf)
    bufs = [rows0, rows1]
    nch = GROWS // GCH
    for k in range(nch):
        rb = bufs[k % 2]
        pltpu.async_copy(z_hbm.at[ibuf.at[pl.ds(k * GCH, GCH)]], rb, sem).wait()
        pltpu.sync_copy(rb, out_hbm.at[pl.ds(base + k * GCH, GCH), :])


def _gather_sc(z, idx):
    mesh = plsc.VectorSubcoreMesh(core_axis_name="c", subcore_axis_name="s")
    f = pl.kernel(
        _gather_body,
        out_type=jax.ShapeDtypeStruct((GB, 128), jnp.float32),
        mesh=mesh,
        scratch_types=[
            pltpu.VMEM((GROWS,), jnp.int32),
            pltpu.VMEM((GCH, 128), jnp.float32),
            pltpu.VMEM((GCH, 128), jnp.float32),
            pltpu.SemaphoreType.DMA,
        ],
        compiler_params=pltpu.CompilerParams(needs_layout_passes=False),
    )
    return f(z, idx)


def _mma_kernel(x_ref, w_ref, b_ref, o_ref, a16_ref, acc_ref, *, nk, first):
    k = pl.program_id(1)

    @pl.when(k == 0)
    def _():
        acc_ref[...] = jnp.zeros_like(acc_ref)

    if first:
        xb = x_ref[...].astype(jnp.bfloat16)
        a16_ref[...] = xb
    else:
        xb = x_ref[...]
    acc_ref[...] += jnp.dot(xb, w_ref[...].astype(jnp.bfloat16),
                            preferred_element_type=jnp.float32)

    @pl.when(k == nk - 1)
    def _():
        res = jnp.maximum(acc_ref[...] + b_ref[...], 0.0)
        i = pl.program_id(0)
        mb, n = acc_ref.shape
        row = i * mb + lax.broadcasted_iota(jnp.int32, (mb, n), 0)
        o_ref[...] = jnp.where(row < N_NODES, res, 0.0)


def _mm_a(x, h, b, first, mb=2048, kb=1024):
    n = h.shape[1]
    b = b.reshape(1, n)
    nk = NP // kb
    grid = (NP // mb, nk)
    out_shape = [jax.ShapeDtypeStruct((NP, n), jnp.float32)]
    out_specs = [pl.BlockSpec((mb, n), lambda i, j: (i, 0))]
    if first:
        out_shape.append(jax.ShapeDtypeStruct((NP, NP), jnp.bfloat16))
        out_specs.append(pl.BlockSpec((mb, kb), lambda i, j: (i, j)))
    else:
        out_shape.append(jax.ShapeDtypeStruct((8, 128), jnp.bfloat16))
        out_specs.append(pl.BlockSpec((8, 128), lambda i, j: (0, 0)))
    res = pl.pallas_call(
        functools.partial(_mma_kernel, nk=nk, first=first),
        grid=grid,
        in_specs=[
            pl.BlockSpec((mb, kb), lambda i, j: (i, j)),
            pl.BlockSpec((kb, n), lambda i, j: (j, 0)),
            pl.BlockSpec((1, n), lambda i, j: (0, 0)),
        ],
        out_specs=out_specs,
        out_shape=out_shape,
        scratch_shapes=[pltpu.VMEM((mb, n), jnp.float32)],
    )(x, h, b)
    return res


def _mm_kernel(x_ref, w_ref, b_ref, o_ref, acc_ref, *, nk, relu, mask_rows):
    k = pl.program_id(1)

    @pl.when(k == 0)
    def _():
        acc_ref[...] = jnp.zeros_like(acc_ref)

    acc_ref[...] += jnp.dot(x_ref[...], w_ref[...],
                            preferred_element_type=jnp.float32)

    @pl.when(k == nk - 1)
    def _():
        res = acc_ref[...] + b_ref[...]
        if relu:
            res = jnp.maximum(res, 0.0)
        if mask_rows:
            i = pl.program_id(0)
            mb, n = acc_ref.shape
            row = i * mb + lax.broadcasted_iota(jnp.int32, (mb, n), 0)
            res = jnp.where(row < N_NODES, res, 0.0)
        o_ref[...] = res


def _mm(x, w, b=None, relu=False, mask_rows=False, mb=2048, kb=1024):
    m, kdim = x.shape
    n = w.shape[1]
    if b is None:
        b = jnp.zeros((1, n), dtype=jnp.float32)
    else:
        b = b.reshape(1, n)
    kb = min(kb, kdim)
    nk = kdim // kb
    grid = (m // mb, nk)
    return pl.pallas_call(
        functools.partial(_mm_kernel, nk=nk, relu=relu, mask_rows=mask_rows),
        grid=grid,
        in_specs=[
            pl.BlockSpec((mb, kb), lambda i, j: (i, j)),
            pl.BlockSpec((kb, n), lambda i, j: (j, 0)),
            pl.BlockSpec((1, n), lambda i, j: (0, 0)),
        ],
        out_specs=pl.BlockSpec((mb, n), lambda i, j: (i, 0)),
        out_shape=jax.ShapeDtypeStruct((m, n), jnp.float32),
        scratch_shapes=[pltpu.VMEM((mb, n), jnp.float32)],
    )(x, w, b)


def _pair_kernel(a_ref, b_ref, o_ref):
    s = jnp.sum(a_ref[...] * b_ref[...], axis=1)
    o_ref[...] = 1.0 / (1.0 + jnp.exp(-s))


def _pair_preds(za, zb):
    m = za.shape[0]
    blk = 8192
    return pl.pallas_call(
        _pair_kernel,
        grid=(m // blk,),
        in_specs=[
            pl.BlockSpec((blk, 128), lambda i: (i, 0)),
            pl.BlockSpec((blk, 128), lambda i: (i, 0)),
        ],
        out_specs=pl.BlockSpec((blk,), lambda i: (i,)),
        out_shape=jax.ShapeDtypeStruct((m,), jnp.float32),
    )(za, zb)


def kernel(x, e, pos, neg, W0, b0, W1, b1, W2, b2, W3, b3, W4, b4, Wc, bc):
    loop = jnp.arange(N_NODES, dtype=e.dtype)
    srcA = jnp.concatenate([e[0], loop])
    dstA = jnp.concatenate([e[1], loop])
    key = dstA.astype(jnp.int32) * NP + srcA.astype(jnp.int32)
    ks = jnp.sort(key)
    dstS = ks // NP
    srcS = ks - dstS * NP
    posi = jnp.arange(EDGES, dtype=jnp.int32)
    isf = jnp.concatenate([jnp.array([True]), ks[1:] != ks[:-1]])
    firsts = jnp.where(isf, posi, jnp.int32(EDGES))
    nxt = lax.cummin(firsts[::-1])[::-1]
    nxt_after = jnp.concatenate([nxt[1:], jnp.array([EDGES], jnp.int32)])
    cf = jnp.where(isf, (nxt_after - posi).astype(jnp.float32), 0.0)
    rowb = jnp.arange(0, NP + ROWS_PER_TILE, ROWS_PER_TILE, dtype=jnp.int32)
    offs = jnp.searchsorted(dstS, rowb[:33], side="left").astype(jnp.int32)
    offs = jnp.concatenate([offs, jnp.zeros((15,), jnp.int32)])
    pad = EALLOC - EDGES
    srcP = jnp.concatenate([srcS, jnp.zeros((pad,), jnp.int32)])
    dstP = jnp.concatenate([dstS, jnp.zeros((pad,), jnp.int32)])
    cfP = jnp.concatenate([cf, jnp.zeros((pad,), jnp.float32)])

    A = _build_adj_sc(srcP, dstP, cfP, offs).reshape(NP, NP)

    z = jnp.zeros((NP, 128), jnp.float32).at[:N_NODES].set(x)
    A16 = None
    for li, (W, b) in enumerate(((W0, b0), (W1, b1), (W2, b2),
                                 (W3, b3), (W4, b4))):
        h = _mm(z, W)
        if li == 0:
            z, A16 = _mm_a(A, h, b, first=True)
        else:
            z, _ = _mm_a(A16, h, b, first=False)

    Wcp = jnp.zeros((128, 128), jnp.float32).at[:, :4].set(Wc)
    bcp = jnp.zeros((128,), jnp.float32).at[:4].set(bc)
    logits = _mm(z, Wcp, b=bcp)[:N_NODES, :4]

    gidx = jnp.concatenate([pos[0], neg[0], pos[1], neg[1]]).astype(jnp.int32)
    rows = _gather_sc(z, gidx)
    preds = _pair_preds(rows[: GB // 2], rows[GB // 2:])
    return (z[:N_NODES], logits, preds)
